# baseline (speedup 1.0000x reference)
import functools

import jax
import jax.numpy as jnp
from jax import lax
from jax.experimental import pallas as pl
from jax.experimental.pallas import tpu as pltpu
from jax.experimental.pallas import tpu_sc as plsc

B = 16384
D = 32
NC = 2
NS = 16
NW = NC * NS
BPW = B // NW
NCHUNK = 4
CHUNK = BPW // NCHUNK


def _body(user_hbm, item_hbm, ut_hbm, it_hbm, out_hbm,
          idx_u, idx_v, rows_u, rows_v, out_v, sem_idx, sem_rows):
    wid = lax.axis_index("s") * NC + lax.axis_index("c")
    base = wid * BPW

    idx_cps = []
    for j in range(NCHUNK):
        idx_cps.append(pltpu.async_copy(
            user_hbm.at[pl.ds(base + j * CHUNK, CHUNK)], idx_u.at[j], sem_idx))
        idx_cps.append(pltpu.async_copy(
            item_hbm.at[pl.ds(base + j * CHUNK, CHUNK)], idx_v.at[j], sem_idx))
    for cp in idx_cps:
        cp.wait()

    row_cps = []
    for j in range(NCHUNK):
        row_cps.append(pltpu.async_copy(
            ut_hbm.at[idx_u.at[j]], rows_u.at[pl.ds(j * CHUNK, CHUNK)],
            sem_rows))
        row_cps.append(pltpu.async_copy(
            it_hbm.at[idx_v.at[j]], rows_v.at[pl.ds(j * CHUNK, CHUNK)],
            sem_rows))
    for cp in row_cps:
        cp.wait()

    lane = lax.iota(jnp.int32, 16)

    def group(g, carry):
        acc = jnp.zeros((16,), jnp.float32)
        for i in range(16):
            r = g * 16 + i
            u0 = rows_u[r, pl.ds(0, 16)]
            u1 = rows_u[r, pl.ds(16, 16)]
            v0 = rows_v[r, pl.ds(0, 16)]
            v1 = rows_v[r, pl.ds(16, 16)]
            total = jnp.sum(u0 * v0 + u1 * v1)
            acc = jnp.where(lane == i, total, acc)
        out_v[pl.ds(g * 16, 16)] = acc
        return carry

    lax.fori_loop(0, BPW // 16, group, 0)
    pltpu.sync_copy(out_v, out_hbm.at[pl.ds(base, BPW)])


@functools.partial(
    pl.kernel,
    out_type=jax.ShapeDtypeStruct((B,), jnp.float32),
    mesh=plsc.VectorSubcoreMesh(core_axis_name="c", subcore_axis_name="s"),
    compiler_params=pltpu.CompilerParams(
        needs_layout_passes=False, use_tc_tiling_on_sc=False),
    scratch_types=[
        pltpu.VMEM((NCHUNK, CHUNK), jnp.int32),
        pltpu.VMEM((NCHUNK, CHUNK), jnp.int32),
        pltpu.VMEM((BPW, D), jnp.float32),
        pltpu.VMEM((BPW, D), jnp.float32),
        pltpu.VMEM((BPW,), jnp.float32),
        pltpu.SemaphoreType.DMA,
        pltpu.SemaphoreType.DMA,
    ],
)
def _dot_kernel(user_hbm, item_hbm, ut_hbm, it_hbm, out_hbm,
                idx_u, idx_v, rows_u, rows_v, out_v, sem_idx, sem_rows):
    _body(user_hbm, item_hbm, ut_hbm, it_hbm, out_hbm,
          idx_u, idx_v, rows_u, rows_v, out_v, sem_idx, sem_rows)


def kernel(user, item, user_table, item_table):
    return _dot_kernel(user.astype(jnp.int32), item.astype(jnp.int32),
                       user_table, item_table)
